# baseline (speedup 1.0000x reference)
import functools

import jax
import jax.numpy as jnp
from jax import lax
from jax.experimental import pallas as pl
from jax.experimental.pallas import tpu as pltpu
from jax.experimental.pallas import tpu_sc as plsc

_NC = 2
_NS = 16
_NW = _NC * _NS
_C = 80
_LANES = 16


def _node_proj_body(x_ref, h_ref, w1a_ref, w1b_ref, b1_ref, a_ref, b_ref):
    a_ref[...] = (
        jnp.dot(x_ref[...], w1a_ref[...], preferred_element_type=jnp.float32)
        + b1_ref[...]
    )
    b_ref[...] = jnp.dot(h_ref[...], w1b_ref[...], preferred_element_type=jnp.float32)


def _mlp_body(e1_ref, w2t_ref, b2_ref, a3_ref, b3_ref, out_ref):
    x = jnp.maximum(e1_ref[...], 0.0)
    h2 = jnp.maximum(
        jnp.dot(x, w2t_ref[...], preferred_element_type=jnp.float32) + b2_ref[...],
        0.0,
    )
    s = jnp.sum(h2 * a3_ref[...], axis=1, keepdims=True) + b3_ref[...]
    out_ref[...] = jnp.exp(s)


def _finish_body(h0_ref, h1_ref, s0_ref, s1_ref, wt_ref, wb_ref, out_ref):
    s = s0_ref[...] + s1_ref[...]
    inv = jnp.where(s > 0.0, 1.0 / s, 0.0)
    h = (h0_ref[...] + h1_ref[...]) * inv
    out_ref[...] = (
        jnp.dot(h, wt_ref[...], preferred_element_type=jnp.float32) + wb_ref[...]
    )


def _edge_gather_body(nchunks, a_hbm, b_hbm, tr_hbm, te_hbm, out_hbm,
                      idx_tr, idx_te, rows_a, rows_b, sem_a, sem_b, sem_w):
    wid = lax.axis_index("s") * _NC + lax.axis_index("c")
    base = wid * (nchunks * _C)

    def start(j, b):
        off = base + j * _C
        pltpu.sync_copy(tr_hbm.at[pl.ds(off, _C)], idx_tr.at[b])
        pltpu.sync_copy(te_hbm.at[pl.ds(off, _C)], idx_te.at[b])
        pltpu.async_copy(a_hbm.at[idx_tr.at[b]], rows_a.at[b], sem_a.at[b])
        pltpu.async_copy(b_hbm.at[idx_te.at[b]], rows_b.at[b], sem_b.at[b])

    start(0, 0)

    def chunk(j, carry):
        b = lax.rem(j, 2)
        nb = 1 - b

        @pl.when(j + 1 < nchunks)
        def _start_next():
            @pl.when(j >= 1)
            def _():
                pltpu.make_async_copy(rows_a.at[nb],
                                      out_hbm.at[pl.ds(base, _C), :],
                                      sem_w.at[nb]).wait()
            start(j + 1, nb)

        pltpu.make_async_copy(a_hbm.at[idx_tr.at[b]], rows_a.at[b],
                              sem_a.at[b]).wait()
        pltpu.make_async_copy(b_hbm.at[idx_te.at[b]], rows_b.at[b],
                              sem_b.at[b]).wait()

        def row(i, c2):
            for v in range(128 // _LANES):
                sl = pl.ds(v * _LANES, _LANES)
                rows_a[b, i, sl] = rows_a[b, i, sl] + rows_b[b, i, sl]
            return c2

        lax.fori_loop(0, _C, row, 0)
        off = base + j * _C
        pltpu.async_copy(rows_a.at[b], out_hbm.at[pl.ds(off, _C), :],
                         sem_w.at[b])
        return carry

    lax.fori_loop(0, nchunks, chunk, 0)
    pltpu.make_async_copy(rows_a.at[0], out_hbm.at[pl.ds(base, _C), :],
                          sem_w.at[0]).wait()
    pltpu.make_async_copy(rows_a.at[1], out_hbm.at[pl.ds(base, _C), :],
                          sem_w.at[1]).wait()


def _scatter_body(nchunks, n_nodes, hi_hbm, tr_hbm, te_hbm, ex_hbm,
                  hs_out, ss_out,
                  hs_sh, ss_sh, idx_tr, idx_te, rows, exbuf, zrows, zscal, sem):
    cid = lax.axis_index("c")
    sid = lax.axis_index("s")
    wid = sid * _NC + cid
    base = wid * (nchunks * _C)

    zr = zrows.shape[0]
    rows_per_cp = n_nodes // 10

    def zrow(i, c):
        for v in range(128 // _LANES):
            zrows[i, pl.ds(v * _LANES, _LANES)] = jnp.zeros((_LANES,), jnp.float32)
        return c
    lax.fori_loop(0, zr, zrow, 0)

    def zsc(i, c):
        zscal[pl.ds(i * _LANES, _LANES)] = jnp.zeros((_LANES,), jnp.float32)
        return c
    lax.fori_loop(0, zscal.shape[0] // _LANES, zsc, 0)

    @pl.when(sid < 10)
    def _zero_hs():
        for k in range(rows_per_cp // zr):
            pltpu.sync_copy(zrows,
                            hs_sh.at[pl.ds(sid * rows_per_cp + k * zr, zr), :])

    @pl.when(sid == 0)
    def _zero_ssum():
        zn = zscal.shape[0]
        for k in range(n_nodes // zn):
            pltpu.sync_copy(zscal, ss_sh.at[pl.ds(k * zn, zn)])

    plsc.subcore_barrier()

    def start(j, b):
        off = base + j * _C
        pltpu.sync_copy(tr_hbm.at[pl.ds(off, _C)], idx_tr.at[b])
        pltpu.sync_copy(te_hbm.at[pl.ds(off, _C)], idx_te.at[b])
        pltpu.sync_copy(ex_hbm.at[pl.ds(off, _C)], exbuf.at[b])
        pltpu.async_copy(hi_hbm.at[idx_tr.at[b]], rows.at[b], sem.at[b])

    start(0, 0)

    dn = lax.GatherDimensionNumbers(offset_dims=(), collapsed_slice_dims=(0,),
                                    start_index_map=(0,))

    def chunk(j, carry):
        b = lax.rem(j, 2)
        nb = 1 - b

        @pl.when(j + 1 < nchunks)
        def _start_next():
            start(j + 1, nb)

        pltpu.make_async_copy(hi_hbm.at[idx_tr.at[b]], rows.at[b],
                              sem.at[b]).wait()

        def row(i, c2):
            g = (i // _LANES) * _LANES
            lane = i - g
            ex16 = exbuf[b, pl.ds(g, _LANES)]
            exv = lax.gather(ex16, jnp.full((_LANES, 1), lane, jnp.int32), dn,
                             (1,), mode=lax.GatherScatterMode.PROMISE_IN_BOUNDS)
            for v in range(128 // _LANES):
                sl = pl.ds(v * _LANES, _LANES)
                rows[b, i, sl] = rows[b, i, sl] * exv
            return c2

        lax.fori_loop(0, _C, row, 0)
        pltpu.sync_copy(rows.at[b], hs_sh.at[idx_te.at[b]], add=True)
        pltpu.sync_copy(exbuf.at[b], ss_sh.at[idx_te.at[b]], add=True)
        return carry

    lax.fori_loop(0, nchunks, chunk, 0)

    plsc.subcore_barrier()

    @pl.when(sid < 10)
    def _copy_out():
        r0 = sid * rows_per_cp
        pltpu.sync_copy(hs_sh.at[pl.ds(r0, rows_per_cp), :],
                        hs_out.at[cid, pl.ds(r0, rows_per_cp), :])
        pltpu.sync_copy(ss_sh.at[pl.ds(r0, rows_per_cp)], zscal)
        pltpu.sync_copy(zscal,
                        ss_out.at[pl.ds(cid * n_nodes + r0, rows_per_cp)])


def kernel(user_feat, hi, edge_index, att1_w, att1_b, att2_w, att2_b,
           att3_w, att3_b, w_w, w_b):
    n, d = user_feat.shape
    e = edge_index.shape[1]
    assert d == 128 and e % (_NW * _C) == 0 and n % _NS == 0 and n % 10 == 0

    trust = edge_index[0].astype(jnp.int32)
    trustee = edge_index[1].astype(jnp.int32)

    w1a_t = att1_w[:, :d].T
    w1b_t = att1_w[:, d:].T
    b1 = att1_b[None, :]
    w2t = att2_w.T
    b2 = att2_b[None, :]
    a3 = att3_w
    b3 = att3_b.reshape(1, 1)
    wwt = w_w.T
    wb = w_b[None, :]

    bn = 1000
    grid_n = n // bn
    f32 = jnp.float32
    a_tab, b_tab = pl.pallas_call(
        _node_proj_body,
        grid=(grid_n,),
        in_specs=[
            pl.BlockSpec((bn, d), lambda i: (i, 0)),
            pl.BlockSpec((bn, d), lambda i: (i, 0)),
            pl.BlockSpec((d, d), lambda i: (0, 0)),
            pl.BlockSpec((d, d), lambda i: (0, 0)),
            pl.BlockSpec((1, d), lambda i: (0, 0)),
        ],
        out_specs=[
            pl.BlockSpec((bn, d), lambda i: (i, 0)),
            pl.BlockSpec((bn, d), lambda i: (i, 0)),
        ],
        out_shape=[
            jax.ShapeDtypeStruct((n, d), f32),
            jax.ShapeDtypeStruct((n, d), f32),
        ],
    )(user_feat, hi, w1a_t, w1b_t, b1)

    nchunks = e // (_NW * _C)
    mesh = plsc.VectorSubcoreMesh(core_axis_name="c", subcore_axis_name="s",
                                  num_cores=_NC, num_subcores=_NS)
    e1 = pl.kernel(
        functools.partial(_edge_gather_body, nchunks),
        out_type=jax.ShapeDtypeStruct((e, d), f32),
        mesh=mesh,
        scratch_types=[
            pltpu.VMEM((2, _C), jnp.int32),
            pltpu.VMEM((2, _C), jnp.int32),
            pltpu.VMEM((2, _C, d), f32),
            pltpu.VMEM((2, _C, d), f32),
            pltpu.SemaphoreType.DMA((2,)),
            pltpu.SemaphoreType.DMA((2,)),
            pltpu.SemaphoreType.DMA((2,)),
        ],
    )(a_tab, b_tab, trust, trustee)

    be = 2560
    grid_e = e // be
    ex = pl.pallas_call(
        _mlp_body,
        grid=(grid_e,),
        in_specs=[
            pl.BlockSpec((be, d), lambda i: (i, 0)),
            pl.BlockSpec((d, d), lambda i: (0, 0)),
            pl.BlockSpec((1, d), lambda i: (0, 0)),
            pl.BlockSpec((1, d), lambda i: (0, 0)),
            pl.BlockSpec((1, 1), lambda i: (0, 0)),
        ],
        out_specs=pl.BlockSpec((be, 1), lambda i: (i, 0)),
        out_shape=jax.ShapeDtypeStruct((e, 1), f32),
    )(e1, w2t, b2, a3, b3)
    ex_flat = ex.reshape(e)

    hs_parts, ss_parts = pl.kernel(
        functools.partial(_scatter_body, nchunks, n),
        out_type=(
            jax.ShapeDtypeStruct((_NC, n, d), f32),
            jax.ShapeDtypeStruct((_NC * n,), f32),
        ),
        mesh=mesh,
        scratch_types=[
            pltpu.VMEM_SHARED((n, d), f32),
            pltpu.VMEM_SHARED((n,), f32),
            pltpu.VMEM((2, _C), jnp.int32),
            pltpu.VMEM((2, _C), jnp.int32),
            pltpu.VMEM((2, _C, d), f32),
            pltpu.VMEM((2, _C), f32),
            pltpu.VMEM((200, d), f32),
            pltpu.VMEM((1000,), f32),
            pltpu.SemaphoreType.DMA((2,)),
        ],
    )(hi, trust, trustee, ex_flat)
    ss_parts = ss_parts.reshape(_NC, n)

    out = pl.pallas_call(
        _finish_body,
        grid=(grid_n,),
        in_specs=[
            pl.BlockSpec((bn, d), lambda i: (i, 0)),
            pl.BlockSpec((bn, d), lambda i: (i, 0)),
            pl.BlockSpec((bn, 1), lambda i: (i, 0)),
            pl.BlockSpec((bn, 1), lambda i: (i, 0)),
            pl.BlockSpec((d, d), lambda i: (0, 0)),
            pl.BlockSpec((1, d), lambda i: (0, 0)),
        ],
        out_specs=pl.BlockSpec((bn, d), lambda i: (i, 0)),
        out_shape=jax.ShapeDtypeStruct((n, d), f32),
    )(hs_parts[0], hs_parts[1], ss_parts[0][:, None], ss_parts[1][:, None],
      wwt, wb)
    return out


# speedup vs baseline: 1.3067x; 1.3067x over previous
import functools

import jax
import jax.numpy as jnp
from jax import lax
from jax.experimental import pallas as pl
from jax.experimental.pallas import tpu as pltpu
from jax.experimental.pallas import tpu_sc as plsc

_NC = 2
_NS = 16
_NW = _NC * _NS
_C = 80
_LANES = 16


def _node_proj_body(x_ref, h_ref, w1a_ref, w1b_ref, b1_ref, a_ref, b_ref):
    a_ref[...] = (
        jnp.dot(x_ref[...], w1a_ref[...], preferred_element_type=jnp.float32)
        + b1_ref[...]
    )
    b_ref[...] = jnp.dot(h_ref[...], w1b_ref[...], preferred_element_type=jnp.float32)


def _mlp_body(g1_ref, g2_ref, g3_ref, w2t_ref, b2_ref, a3_ref, b3_ref,
              m_ref, ex_ref):
    x = jnp.maximum(g1_ref[...] + g2_ref[...], 0.0)
    h2 = jnp.maximum(
        jnp.dot(x, w2t_ref[...], preferred_element_type=jnp.float32) + b2_ref[...],
        0.0,
    )
    s = jnp.sum(h2 * a3_ref[...], axis=1, keepdims=True) + b3_ref[...]
    e = jnp.exp(s)
    ex_ref[...] = e
    m_ref[...] = e * g3_ref[...]


def _finish_body(h0_ref, h1_ref, s0_ref, s1_ref, wt_ref, wb_ref, out_ref):
    s = s0_ref[...] + s1_ref[...]
    inv = jnp.where(s > 0.0, 1.0 / s, 0.0)
    h = (h0_ref[...] + h1_ref[...]) * inv
    out_ref[...] = (
        jnp.dot(h, wt_ref[...], preferred_element_type=jnp.float32) + wb_ref[...]
    )


def _edge_gather_body(nchunks, a_hbm, b_hbm, h_hbm, tr_hbm, te_hbm,
                      ga_hbm, gb_hbm, gh_hbm,
                      idx_tr, idx_te, rows_a, rows_b, rows_h,
                      sem_a, sem_b, sem_h, sw_a, sw_b, sw_h):
    wid = lax.axis_index("s") * _NC + lax.axis_index("c")
    base = wid * (nchunks * _C)

    def start(j, b):
        off = base + j * _C
        pltpu.sync_copy(tr_hbm.at[pl.ds(off, _C)], idx_tr.at[b])
        pltpu.sync_copy(te_hbm.at[pl.ds(off, _C)], idx_te.at[b])
        pltpu.async_copy(a_hbm.at[idx_tr.at[b]], rows_a.at[b], sem_a.at[b])
        pltpu.async_copy(b_hbm.at[idx_te.at[b]], rows_b.at[b], sem_b.at[b])
        pltpu.async_copy(h_hbm.at[idx_tr.at[b]], rows_h.at[b], sem_h.at[b])

    def wait_wb(b):
        sl = pl.ds(base, _C)
        pltpu.make_async_copy(rows_a.at[b], ga_hbm.at[sl, :], sw_a.at[b]).wait()
        pltpu.make_async_copy(rows_b.at[b], gb_hbm.at[sl, :], sw_b.at[b]).wait()
        pltpu.make_async_copy(rows_h.at[b], gh_hbm.at[sl, :], sw_h.at[b]).wait()

    start(0, 0)

    def chunk(j, carry):
        b = lax.rem(j, 2)
        nb = 1 - b

        @pl.when(j + 1 < nchunks)
        def _start_next():
            @pl.when(j >= 1)
            def _():
                wait_wb(nb)
            start(j + 1, nb)

        pltpu.make_async_copy(a_hbm.at[idx_tr.at[b]], rows_a.at[b],
                              sem_a.at[b]).wait()
        pltpu.make_async_copy(b_hbm.at[idx_te.at[b]], rows_b.at[b],
                              sem_b.at[b]).wait()
        pltpu.make_async_copy(h_hbm.at[idx_tr.at[b]], rows_h.at[b],
                              sem_h.at[b]).wait()
        off = base + j * _C
        pltpu.async_copy(rows_a.at[b], ga_hbm.at[pl.ds(off, _C), :], sw_a.at[b])
        pltpu.async_copy(rows_b.at[b], gb_hbm.at[pl.ds(off, _C), :], sw_b.at[b])
        pltpu.async_copy(rows_h.at[b], gh_hbm.at[pl.ds(off, _C), :], sw_h.at[b])
        return carry

    lax.fori_loop(0, nchunks, chunk, 0)
    wait_wb(0)
    wait_wb(1)


def _scatter_body(nchunks, n_nodes, m_hbm, te_hbm, ex_hbm,
                  hs_out, ss_out,
                  hs_sh, ss_sh, idx_te, rows, exbuf, zrows, zscal, sem):
    cid = lax.axis_index("c")
    sid = lax.axis_index("s")
    wid = sid * _NC + cid
    base = wid * (nchunks * _C)

    zr = zrows.shape[0]
    rows_per_cp = n_nodes // 10

    def zrow(i, c):
        for v in range(128 // _LANES):
            zrows[i, pl.ds(v * _LANES, _LANES)] = jnp.zeros((_LANES,), jnp.float32)
        return c
    lax.fori_loop(0, zr, zrow, 0)

    def zsc(i, c):
        zscal[pl.ds(i * _LANES, _LANES)] = jnp.zeros((_LANES,), jnp.float32)
        return c
    lax.fori_loop(0, zscal.shape[0] // _LANES, zsc, 0)

    @pl.when(sid < 10)
    def _zero_hs():
        for k in range(rows_per_cp // zr):
            pltpu.sync_copy(zrows,
                            hs_sh.at[pl.ds(sid * rows_per_cp + k * zr, zr), :])

    @pl.when(sid == 0)
    def _zero_ssum():
        zn = zscal.shape[0]
        for k in range(n_nodes // zn):
            pltpu.sync_copy(zscal, ss_sh.at[pl.ds(k * zn, zn)])

    plsc.subcore_barrier()

    def start(j, b):
        off = base + j * _C
        pltpu.sync_copy(te_hbm.at[pl.ds(off, _C)], idx_te.at[b])
        pltpu.sync_copy(ex_hbm.at[pl.ds(off, _C)], exbuf.at[b])
        pltpu.async_copy(m_hbm.at[pl.ds(off, _C), :], rows.at[b], sem.at[b])

    start(0, 0)

    def chunk(j, carry):
        b = lax.rem(j, 2)
        nb = 1 - b

        @pl.when(j + 1 < nchunks)
        def _start_next():
            start(j + 1, nb)

        pltpu.make_async_copy(m_hbm.at[pl.ds(base, _C), :], rows.at[b],
                              sem.at[b]).wait()
        pltpu.sync_copy(rows.at[b], hs_sh.at[idx_te.at[b]], add=True)
        pltpu.sync_copy(exbuf.at[b], ss_sh.at[idx_te.at[b]], add=True)
        return carry

    lax.fori_loop(0, nchunks, chunk, 0)

    plsc.subcore_barrier()

    @pl.when(sid < 10)
    def _copy_out():
        r0 = sid * rows_per_cp
        pltpu.sync_copy(hs_sh.at[pl.ds(r0, rows_per_cp), :],
                        hs_out.at[cid, pl.ds(r0, rows_per_cp), :])
        pltpu.sync_copy(ss_sh.at[pl.ds(r0, rows_per_cp)], zscal)
        pltpu.sync_copy(zscal,
                        ss_out.at[pl.ds(cid * n_nodes + r0, rows_per_cp)])


def kernel(user_feat, hi, edge_index, att1_w, att1_b, att2_w, att2_b,
           att3_w, att3_b, w_w, w_b):
    n, d = user_feat.shape
    e = edge_index.shape[1]
    assert d == 128 and e % (_NW * _C) == 0 and n % _NS == 0 and n % 10 == 0

    trust = edge_index[0].astype(jnp.int32)
    trustee = edge_index[1].astype(jnp.int32)

    w1a_t = att1_w[:, :d].T
    w1b_t = att1_w[:, d:].T
    b1 = att1_b[None, :]
    w2t = att2_w.T
    b2 = att2_b[None, :]
    a3 = att3_w
    b3 = att3_b.reshape(1, 1)
    wwt = w_w.T
    wb = w_b[None, :]

    bn = 1000
    grid_n = n // bn
    f32 = jnp.float32
    a_tab, b_tab = pl.pallas_call(
        _node_proj_body,
        grid=(grid_n,),
        in_specs=[
            pl.BlockSpec((bn, d), lambda i: (i, 0)),
            pl.BlockSpec((bn, d), lambda i: (i, 0)),
            pl.BlockSpec((d, d), lambda i: (0, 0)),
            pl.BlockSpec((d, d), lambda i: (0, 0)),
            pl.BlockSpec((1, d), lambda i: (0, 0)),
        ],
        out_specs=[
            pl.BlockSpec((bn, d), lambda i: (i, 0)),
            pl.BlockSpec((bn, d), lambda i: (i, 0)),
        ],
        out_shape=[
            jax.ShapeDtypeStruct((n, d), f32),
            jax.ShapeDtypeStruct((n, d), f32),
        ],
    )(user_feat, hi, w1a_t, w1b_t, b1)

    nchunks = e // (_NW * _C)
    mesh = plsc.VectorSubcoreMesh(core_axis_name="c", subcore_axis_name="s",
                                  num_cores=_NC, num_subcores=_NS)
    g1, g2, g3 = pl.kernel(
        functools.partial(_edge_gather_body, nchunks),
        out_type=(
            jax.ShapeDtypeStruct((e, d), f32),
            jax.ShapeDtypeStruct((e, d), f32),
            jax.ShapeDtypeStruct((e, d), f32),
        ),
        mesh=mesh,
        scratch_types=[
            pltpu.VMEM((2, _C), jnp.int32),
            pltpu.VMEM((2, _C), jnp.int32),
            pltpu.VMEM((2, _C, d), f32),
            pltpu.VMEM((2, _C, d), f32),
            pltpu.VMEM((2, _C, d), f32),
            pltpu.SemaphoreType.DMA((2,)),
            pltpu.SemaphoreType.DMA((2,)),
            pltpu.SemaphoreType.DMA((2,)),
            pltpu.SemaphoreType.DMA((2,)),
            pltpu.SemaphoreType.DMA((2,)),
            pltpu.SemaphoreType.DMA((2,)),
        ],
    )(a_tab, b_tab, hi, trust, trustee)

    be = 2560
    grid_e = e // be
    msg, ex = pl.pallas_call(
        _mlp_body,
        grid=(grid_e,),
        in_specs=[
            pl.BlockSpec((be, d), lambda i: (i, 0)),
            pl.BlockSpec((be, d), lambda i: (i, 0)),
            pl.BlockSpec((be, d), lambda i: (i, 0)),
            pl.BlockSpec((d, d), lambda i: (0, 0)),
            pl.BlockSpec((1, d), lambda i: (0, 0)),
            pl.BlockSpec((1, d), lambda i: (0, 0)),
            pl.BlockSpec((1, 1), lambda i: (0, 0)),
        ],
        out_specs=[
            pl.BlockSpec((be, d), lambda i: (i, 0)),
            pl.BlockSpec((be, 1), lambda i: (i, 0)),
        ],
        out_shape=[
            jax.ShapeDtypeStruct((e, d), f32),
            jax.ShapeDtypeStruct((e, 1), f32),
        ],
    )(g1, g2, g3, w2t, b2, a3, b3)
    ex_flat = ex.reshape(e)

    hs_parts, ss_parts = pl.kernel(
        functools.partial(_scatter_body, nchunks, n),
        out_type=(
            jax.ShapeDtypeStruct((_NC, n, d), f32),
            jax.ShapeDtypeStruct((_NC * n,), f32),
        ),
        mesh=mesh,
        scratch_types=[
            pltpu.VMEM_SHARED((n, d), f32),
            pltpu.VMEM_SHARED((n,), f32),
            pltpu.VMEM((2, _C), jnp.int32),
            pltpu.VMEM((2, _C, d), f32),
            pltpu.VMEM((2, _C), f32),
            pltpu.VMEM((200, d), f32),
            pltpu.VMEM((1000,), f32),
            pltpu.SemaphoreType.DMA((2,)),
        ],
    )(msg, trustee, ex_flat)
    ss_parts = ss_parts.reshape(_NC, n)

    out = pl.pallas_call(
        _finish_body,
        grid=(grid_n,),
        in_specs=[
            pl.BlockSpec((bn, d), lambda i: (i, 0)),
            pl.BlockSpec((bn, d), lambda i: (i, 0)),
            pl.BlockSpec((bn, 1), lambda i: (i, 0)),
            pl.BlockSpec((bn, 1), lambda i: (i, 0)),
            pl.BlockSpec((d, d), lambda i: (0, 0)),
            pl.BlockSpec((1, d), lambda i: (0, 0)),
        ],
        out_specs=pl.BlockSpec((bn, d), lambda i: (i, 0)),
        out_shape=jax.ShapeDtypeStruct((n, d), f32),
    )(hs_parts[0], hs_parts[1], ss_parts[0][:, None], ss_parts[1][:, None],
      wwt, wb)
    return out


# speedup vs baseline: 1.4598x; 1.1171x over previous
import functools

import jax
import jax.numpy as jnp
from jax import lax
from jax.experimental import pallas as pl
from jax.experimental.pallas import tpu as pltpu
from jax.experimental.pallas import tpu_sc as plsc

_NC = 2
_NS = 16
_NW = _NC * _NS
_C = 80
_LANES = 16


def _pack_bf16_pair(x):
    u = lax.bitcast_convert_type(x.astype(jnp.bfloat16), jnp.uint16)
    half = x.shape[1] // 2
    lo = u[:, :half].astype(jnp.uint32)
    hi = u[:, half:].astype(jnp.uint32)
    return lo | (hi << 16)


def _unpack_bf16_pair(p):
    lo = lax.bitcast_convert_type((p & 0xFFFF).astype(jnp.uint16),
                                  jnp.bfloat16).astype(jnp.float32)
    hi = lax.bitcast_convert_type((p >> 16).astype(jnp.uint16),
                                  jnp.bfloat16).astype(jnp.float32)
    return lo, hi


def _node_proj_body(x_ref, h_ref, w1a_ref, w1b_ref, b1_ref, ttr_ref, b_ref):
    a = (jnp.dot(x_ref[...], w1a_ref[...], preferred_element_type=jnp.float32)
         + b1_ref[...])
    hd = ttr_ref.shape[1] // 2
    ttr_ref[:, :hd] = _pack_bf16_pair(a)
    ttr_ref[:, hd:] = _pack_bf16_pair(h_ref[...])
    b_ref[...] = jnp.dot(h_ref[...], w1b_ref[...],
                         preferred_element_type=jnp.float32)


def _mlp_body(g1_ref, g2_ref, w2t_ref, b2_ref, a3_ref, b3_ref,
              m_ref, ex_ref):
    g = g1_ref[...]
    hd = g.shape[1] // 2
    a_lo, a_hi = _unpack_bf16_pair(g[:, :hd])
    b = g2_ref[...]
    x_lo = jnp.maximum(a_lo + b[:, :hd], 0.0)
    x_hi = jnp.maximum(a_hi + b[:, hd:], 0.0)
    w2t = w2t_ref[...]
    half = w2t.shape[0] // 2
    h2 = jnp.maximum(
        jnp.dot(x_lo, w2t[:half, :], preferred_element_type=jnp.float32)
        + jnp.dot(x_hi, w2t[half:, :], preferred_element_type=jnp.float32)
        + b2_ref[...],
        0.0,
    )
    s = jnp.sum(h2 * a3_ref[...], axis=1, keepdims=True) + b3_ref[...]
    e = jnp.exp(s)
    ex_ref[...] = e
    h_lo, h_hi = _unpack_bf16_pair(g[:, hd:])
    m_ref[:, :hd] = e * h_lo
    m_ref[:, hd:] = e * h_hi


def _finish_body(h0_ref, h1_ref, s0_ref, s1_ref, wt_ref, wb_ref, out_ref):
    s = s0_ref[...] + s1_ref[...]
    inv = jnp.where(s > 0.0, 1.0 / s, 0.0)
    h = (h0_ref[...] + h1_ref[...]) * inv
    out_ref[...] = (
        jnp.dot(h, wt_ref[...], preferred_element_type=jnp.float32) + wb_ref[...]
    )


def _edge_gather_body(nchunks, a_hbm, b_hbm, tr_hbm, te_hbm,
                      ga_hbm, gb_hbm,
                      idx_tr, idx_te, rows_a, rows_b,
                      sem_a, sem_b, sw_a, sw_b):
    wid = lax.axis_index("s") * _NC + lax.axis_index("c")
    base = wid * (nchunks * _C)

    def start(j, b):
        off = base + j * _C
        pltpu.sync_copy(tr_hbm.at[pl.ds(off, _C)], idx_tr.at[b])
        pltpu.sync_copy(te_hbm.at[pl.ds(off, _C)], idx_te.at[b])
        pltpu.async_copy(a_hbm.at[idx_tr.at[b]], rows_a.at[b], sem_a.at[b])
        pltpu.async_copy(b_hbm.at[idx_te.at[b]], rows_b.at[b], sem_b.at[b])

    def wait_wb(b):
        sl = pl.ds(base, _C)
        pltpu.make_async_copy(rows_a.at[b], ga_hbm.at[sl, :], sw_a.at[b]).wait()
        pltpu.make_async_copy(rows_b.at[b], gb_hbm.at[sl, :], sw_b.at[b]).wait()

    start(0, 0)

    def chunk(j, carry):
        b = lax.rem(j, 2)
        nb = 1 - b

        @pl.when(j + 1 < nchunks)
        def _start_next():
            @pl.when(j >= 1)
            def _():
                wait_wb(nb)
            start(j + 1, nb)

        pltpu.make_async_copy(a_hbm.at[idx_tr.at[b]], rows_a.at[b],
                              sem_a.at[b]).wait()
        pltpu.make_async_copy(b_hbm.at[idx_te.at[b]], rows_b.at[b],
                              sem_b.at[b]).wait()
        off = base + j * _C
        pltpu.async_copy(rows_a.at[b], ga_hbm.at[pl.ds(off, _C), :], sw_a.at[b])
        pltpu.async_copy(rows_b.at[b], gb_hbm.at[pl.ds(off, _C), :], sw_b.at[b])
        return carry

    lax.fori_loop(0, nchunks, chunk, 0)
    wait_wb(0)
    wait_wb(1)


def _scatter_body(nchunks, n_nodes, m_hbm, te_hbm, ex_hbm,
                  hs_out, ss_out,
                  hs_sh, ss_sh, idx_te, rows, exbuf, zrows, zscal, sem):
    cid = lax.axis_index("c")
    sid = lax.axis_index("s")
    wid = sid * _NC + cid
    base = wid * (nchunks * _C)

    zr = zrows.shape[0]
    rows_per_cp = n_nodes // 10

    def zrow(i, c):
        for v in range(128 // _LANES):
            zrows[i, pl.ds(v * _LANES, _LANES)] = jnp.zeros((_LANES,), jnp.float32)
        return c
    lax.fori_loop(0, zr, zrow, 0)

    def zsc(i, c):
        zscal[pl.ds(i * _LANES, _LANES)] = jnp.zeros((_LANES,), jnp.float32)
        return c
    lax.fori_loop(0, zscal.shape[0] // _LANES, zsc, 0)

    @pl.when(sid < 10)
    def _zero_hs():
        for k in range(rows_per_cp // zr):
            pltpu.sync_copy(zrows,
                            hs_sh.at[pl.ds(sid * rows_per_cp + k * zr, zr), :])

    @pl.when(sid == 0)
    def _zero_ssum():
        zn = zscal.shape[0]
        for k in range(n_nodes // zn):
            pltpu.sync_copy(zscal, ss_sh.at[pl.ds(k * zn, zn)])

    plsc.subcore_barrier()

    def start(j, b):
        off = base + j * _C
        pltpu.sync_copy(te_hbm.at[pl.ds(off, _C)], idx_te.at[b])
        pltpu.sync_copy(ex_hbm.at[pl.ds(off, _C)], exbuf.at[b])
        pltpu.async_copy(m_hbm.at[pl.ds(off, _C), :], rows.at[b], sem.at[b])

    start(0, 0)

    def chunk(j, carry):
        b = lax.rem(j, 2)
        nb = 1 - b

        @pl.when(j + 1 < nchunks)
        def _start_next():
            start(j + 1, nb)

        pltpu.make_async_copy(m_hbm.at[pl.ds(base, _C), :], rows.at[b],
                              sem.at[b]).wait()
        pltpu.sync_copy(rows.at[b], hs_sh.at[idx_te.at[b]], add=True)
        pltpu.sync_copy(exbuf.at[b], ss_sh.at[idx_te.at[b]], add=True)
        return carry

    lax.fori_loop(0, nchunks, chunk, 0)

    plsc.subcore_barrier()

    @pl.when(sid < 10)
    def _copy_out():
        r0 = sid * rows_per_cp
        pltpu.sync_copy(hs_sh.at[pl.ds(r0, rows_per_cp), :],
                        hs_out.at[cid, pl.ds(r0, rows_per_cp), :])
        pltpu.sync_copy(ss_sh.at[pl.ds(r0, rows_per_cp)], zscal)
        pltpu.sync_copy(zscal,
                        ss_out.at[pl.ds(cid * n_nodes + r0, rows_per_cp)])


def kernel(user_feat, hi, edge_index, att1_w, att1_b, att2_w, att2_b,
           att3_w, att3_b, w_w, w_b):
    n, d = user_feat.shape
    e = edge_index.shape[1]
    assert d == 128 and e % (_NW * _C) == 0 and n % _NS == 0 and n % 10 == 0

    trust = edge_index[0].astype(jnp.int32)
    trustee = edge_index[1].astype(jnp.int32)

    w1a_t = att1_w[:, :d].T
    w1b_t = att1_w[:, d:].T
    b1 = att1_b[None, :]
    w2t = att2_w.T
    b2 = att2_b[None, :]
    a3 = att3_w
    b3 = att3_b.reshape(1, 1)
    wwt = w_w.T
    wb = w_b[None, :]

    bn = 1000
    grid_n = n // bn
    f32 = jnp.float32
    ttr_tab, b_tab = pl.pallas_call(
        _node_proj_body,
        grid=(grid_n,),
        in_specs=[
            pl.BlockSpec((bn, d), lambda i: (i, 0)),
            pl.BlockSpec((bn, d), lambda i: (i, 0)),
            pl.BlockSpec((d, d), lambda i: (0, 0)),
            pl.BlockSpec((d, d), lambda i: (0, 0)),
            pl.BlockSpec((1, d), lambda i: (0, 0)),
        ],
        out_specs=[
            pl.BlockSpec((bn, d), lambda i: (i, 0)),
            pl.BlockSpec((bn, d), lambda i: (i, 0)),
        ],
        out_shape=[
            jax.ShapeDtypeStruct((n, d), jnp.uint32),
            jax.ShapeDtypeStruct((n, d), f32),
        ],
    )(user_feat, hi, w1a_t, w1b_t, b1)

    nchunks = e // (_NW * _C)
    mesh = plsc.VectorSubcoreMesh(core_axis_name="c", subcore_axis_name="s",
                                  num_cores=_NC, num_subcores=_NS)
    g1, g2 = pl.kernel(
        functools.partial(_edge_gather_body, nchunks),
        out_type=(
            jax.ShapeDtypeStruct((e, d), jnp.uint32),
            jax.ShapeDtypeStruct((e, d), f32),
        ),
        mesh=mesh,
        scratch_types=[
            pltpu.VMEM((2, _C), jnp.int32),
            pltpu.VMEM((2, _C), jnp.int32),
            pltpu.VMEM((2, _C, d), jnp.uint32),
            pltpu.VMEM((2, _C, d), f32),
            pltpu.SemaphoreType.DMA((2,)),
            pltpu.SemaphoreType.DMA((2,)),
            pltpu.SemaphoreType.DMA((2,)),
            pltpu.SemaphoreType.DMA((2,)),
        ],
    )(ttr_tab, b_tab, trust, trustee)

    be = 2560
    grid_e = e // be
    msg, ex = pl.pallas_call(
        _mlp_body,
        grid=(grid_e,),
        in_specs=[
            pl.BlockSpec((be, d), lambda i: (i, 0)),
            pl.BlockSpec((be, d), lambda i: (i, 0)),
            pl.BlockSpec((d, d), lambda i: (0, 0)),
            pl.BlockSpec((1, d), lambda i: (0, 0)),
            pl.BlockSpec((1, d), lambda i: (0, 0)),
            pl.BlockSpec((1, 1), lambda i: (0, 0)),
        ],
        out_specs=[
            pl.BlockSpec((be, d), lambda i: (i, 0)),
            pl.BlockSpec((be, 1), lambda i: (i, 0)),
        ],
        out_shape=[
            jax.ShapeDtypeStruct((e, d), f32),
            jax.ShapeDtypeStruct((e, 1), f32),
        ],
    )(g1, g2, w2t, b2, a3, b3)
    ex_flat = ex.reshape(e)

    hs_parts, ss_parts = pl.kernel(
        functools.partial(_scatter_body, nchunks, n),
        out_type=(
            jax.ShapeDtypeStruct((_NC, n, d), f32),
            jax.ShapeDtypeStruct((_NC * n,), f32),
        ),
        mesh=mesh,
        scratch_types=[
            pltpu.VMEM_SHARED((n, d), f32),
            pltpu.VMEM_SHARED((n,), f32),
            pltpu.VMEM((2, _C), jnp.int32),
            pltpu.VMEM((2, _C, d), f32),
            pltpu.VMEM((2, _C), f32),
            pltpu.VMEM((200, d), f32),
            pltpu.VMEM((1000,), f32),
            pltpu.SemaphoreType.DMA((2,)),
        ],
    )(msg, trustee, ex_flat)
    ss_parts = ss_parts.reshape(_NC, n)

    out = pl.pallas_call(
        _finish_body,
        grid=(grid_n,),
        in_specs=[
            pl.BlockSpec((bn, d), lambda i: (i, 0)),
            pl.BlockSpec((bn, d), lambda i: (i, 0)),
            pl.BlockSpec((bn, 1), lambda i: (i, 0)),
            pl.BlockSpec((bn, 1), lambda i: (i, 0)),
            pl.BlockSpec((d, d), lambda i: (0, 0)),
            pl.BlockSpec((1, d), lambda i: (0, 0)),
        ],
        out_specs=pl.BlockSpec((bn, d), lambda i: (i, 0)),
        out_shape=jax.ShapeDtypeStruct((n, d), f32),
    )(hs_parts[0], hs_parts[1], ss_parts[0][:, None], ss_parts[1][:, None],
      wwt, wb)
    return out


# speedup vs baseline: 1.7161x; 1.1756x over previous
import functools

import jax
import jax.numpy as jnp
from jax import lax
from jax.experimental import pallas as pl
from jax.experimental.pallas import tpu as pltpu
from jax.experimental.pallas import tpu_sc as plsc

_NC = 2
_NS = 16
_NW = _NC * _NS
_C = 80
_LANES = 16


def _pack_bf16_pair(x):
    u = lax.bitcast_convert_type(x.astype(jnp.bfloat16), jnp.uint16)
    half = x.shape[1] // 2
    lo = u[:, :half].astype(jnp.uint32)
    hi = u[:, half:].astype(jnp.uint32)
    return lo | (hi << 16)


def _unpack_bf16_pair(p):
    lo = lax.bitcast_convert_type((p & 0xFFFF).astype(jnp.uint16),
                                  jnp.bfloat16).astype(jnp.float32)
    hi = lax.bitcast_convert_type((p >> 16).astype(jnp.uint16),
                                  jnp.bfloat16).astype(jnp.float32)
    return lo, hi


def _node_proj_body(x_ref, h_ref, w1a_ref, w1b_ref, b1_ref, ttr_ref, b_ref):
    a = (jnp.dot(x_ref[...], w1a_ref[...], preferred_element_type=jnp.float32)
         + b1_ref[...])
    hd = ttr_ref.shape[1] // 2
    ttr_ref[:, :hd] = _pack_bf16_pair(a)
    ttr_ref[:, hd:] = _pack_bf16_pair(h_ref[...])
    b_ref[...] = jnp.dot(h_ref[...], w1b_ref[...],
                         preferred_element_type=jnp.float32)


def _mlp_body(g1_ref, g2_ref, w2t_ref, b2_ref, a3_ref, b3_ref,
              m_ref, ex_ref):
    g = g1_ref[...]
    hd = g.shape[1] // 2
    a_lo, a_hi = _unpack_bf16_pair(g[:, :hd])
    b = g2_ref[...]
    x_lo = jnp.maximum(a_lo + b[:, :hd], 0.0)
    x_hi = jnp.maximum(a_hi + b[:, hd:], 0.0)
    w2t = w2t_ref[...]
    half = w2t.shape[0] // 2
    h2 = jnp.maximum(
        jnp.dot(x_lo, w2t[:half, :], preferred_element_type=jnp.float32)
        + jnp.dot(x_hi, w2t[half:, :], preferred_element_type=jnp.float32)
        + b2_ref[...],
        0.0,
    )
    s = jnp.sum(h2 * a3_ref[...], axis=1, keepdims=True) + b3_ref[...]
    e = jnp.exp(s)
    ex_ref[...] = e
    h_lo, h_hi = _unpack_bf16_pair(g[:, hd:])
    m_ref[:, :hd] = e * h_lo
    m_ref[:, hd:] = e * h_hi


def _finish_body(h0_ref, h1_ref, s0_ref, s1_ref, wt_ref, wb_ref, out_ref):
    s = s0_ref[...] + s1_ref[...]
    inv = jnp.where(s > 0.0, 1.0 / s, 0.0)
    h = (h0_ref[...] + h1_ref[...]) * inv
    out_ref[...] = (
        jnp.dot(h, wt_ref[...], preferred_element_type=jnp.float32) + wb_ref[...]
    )


def _edge_gather_body(nchunks, a_hbm, b_hbm, tr_hbm, te_hbm,
                      ga_hbm, gb_hbm,
                      idx_tr, idx_te, rows_a, rows_b,
                      sem_a, sem_b, sw_a, sw_b):
    wid = lax.axis_index("s") * _NC + lax.axis_index("c")
    pw = nchunks * _C
    base = wid * pw

    pltpu.sync_copy(tr_hbm.at[wid], idx_tr)
    pltpu.sync_copy(te_hbm.at[wid], idx_te)

    def start(j, b):
        pltpu.async_copy(a_hbm.at[idx_tr.at[j]], rows_a.at[b], sem_a.at[b])
        pltpu.async_copy(b_hbm.at[idx_te.at[j]], rows_b.at[b], sem_b.at[b])

    def wait_wb(b):
        sl = pl.ds(base, _C)
        pltpu.make_async_copy(rows_a.at[b], ga_hbm.at[sl, :], sw_a.at[b]).wait()
        pltpu.make_async_copy(rows_b.at[b], gb_hbm.at[sl, :], sw_b.at[b]).wait()

    start(0, 0)

    def chunk(j, carry):
        b = lax.rem(j, 2)
        nb = 1 - b

        @pl.when(j + 1 < nchunks)
        def _start_next():
            @pl.when(j >= 1)
            def _():
                wait_wb(nb)
            start(j + 1, nb)

        pltpu.make_async_copy(a_hbm.at[idx_tr.at[j]], rows_a.at[b],
                              sem_a.at[b]).wait()
        pltpu.make_async_copy(b_hbm.at[idx_te.at[j]], rows_b.at[b],
                              sem_b.at[b]).wait()
        off = base + j * _C
        pltpu.async_copy(rows_a.at[b], ga_hbm.at[pl.ds(off, _C), :], sw_a.at[b])
        pltpu.async_copy(rows_b.at[b], gb_hbm.at[pl.ds(off, _C), :], sw_b.at[b])
        return carry

    lax.fori_loop(0, nchunks, chunk, 0)
    wait_wb(0)
    wait_wb(1)


def _scatter_body(nchunks, n_nodes, m_hbm, te_hbm, ex_hbm,
                  hs_out, ss_out,
                  hs_sh, ss_sh, idx_te, rows, exbuf, zscal, sem):
    cid = lax.axis_index("c")
    sid = lax.axis_index("s")
    wid = sid * _NC + cid
    base = wid * (nchunks * _C)

    rows_per_cp = n_nodes // 10

    def zrow(i, c):
        for v in range(128 // _LANES):
            rows[0, i, pl.ds(v * _LANES, _LANES)] = jnp.zeros((_LANES,),
                                                              jnp.float32)
        return c
    lax.fori_loop(0, _C, zrow, 0)

    def zsc(i, c):
        zscal[pl.ds(i * _LANES, _LANES)] = jnp.zeros((_LANES,), jnp.float32)
        return c
    lax.fori_loop(0, zscal.shape[0] // _LANES, zsc, 0)

    @pl.when(sid < 10)
    def _zero_hs():
        for k in range(rows_per_cp // _C):
            pltpu.sync_copy(rows.at[0],
                            hs_sh.at[pl.ds(sid * rows_per_cp + k * _C, _C), :])
        rem = rows_per_cp % _C
        if rem:
            pltpu.sync_copy(
                rows.at[0, pl.ds(0, rem), :],
                hs_sh.at[pl.ds(sid * rows_per_cp + rows_per_cp - rem, rem), :])

    @pl.when(sid == 0)
    def _zero_ssum():
        zn = zscal.shape[0]
        for k in range(n_nodes // zn):
            pltpu.sync_copy(zscal, ss_sh.at[pl.ds(k * zn, zn)])

    pltpu.sync_copy(te_hbm.at[wid], idx_te)
    pltpu.sync_copy(ex_hbm.at[pl.ds(base, nchunks * _C)], exbuf)

    plsc.subcore_barrier()

    def start(j, b):
        off = base + j * _C
        pltpu.async_copy(m_hbm.at[pl.ds(off, _C), :], rows.at[b], sem.at[b])

    start(0, 0)

    def chunk(j, carry):
        b = lax.rem(j, 2)
        nb = 1 - b

        @pl.when(j + 1 < nchunks)
        def _start_next():
            start(j + 1, nb)

        pltpu.make_async_copy(m_hbm.at[pl.ds(base, _C), :], rows.at[b],
                              sem.at[b]).wait()
        pltpu.sync_copy(rows.at[b], hs_sh.at[idx_te.at[j]], add=True)
        pltpu.sync_copy(exbuf.at[pl.ds(j * _C, _C)], ss_sh.at[idx_te.at[j]],
                        add=True)
        return carry

    lax.fori_loop(0, nchunks, chunk, 0)

    plsc.subcore_barrier()

    @pl.when(sid < 10)
    def _copy_out():
        r0 = sid * rows_per_cp
        pltpu.sync_copy(hs_sh.at[pl.ds(r0, rows_per_cp), :],
                        hs_out.at[cid, pl.ds(r0, rows_per_cp), :])
        pltpu.sync_copy(ss_sh.at[pl.ds(r0, rows_per_cp)], zscal)
        pltpu.sync_copy(zscal,
                        ss_out.at[pl.ds(cid * n_nodes + r0, rows_per_cp)])


def kernel(user_feat, hi, edge_index, att1_w, att1_b, att2_w, att2_b,
           att3_w, att3_b, w_w, w_b):
    n, d = user_feat.shape
    e = edge_index.shape[1]
    assert d == 128 and e % (_NW * _C) == 0 and n % _NS == 0 and n % 10 == 0

    trust = edge_index[0].astype(jnp.int32)
    trustee = edge_index[1].astype(jnp.int32)

    w1a_t = att1_w[:, :d].T
    w1b_t = att1_w[:, d:].T
    b1 = att1_b[None, :]
    w2t = att2_w.T
    b2 = att2_b[None, :]
    a3 = att3_w
    b3 = att3_b.reshape(1, 1)
    wwt = w_w.T
    wb = w_b[None, :]

    bn = 1000
    grid_n = n // bn
    f32 = jnp.float32
    ttr_tab, b_tab = pl.pallas_call(
        _node_proj_body,
        grid=(grid_n,),
        in_specs=[
            pl.BlockSpec((bn, d), lambda i: (i, 0)),
            pl.BlockSpec((bn, d), lambda i: (i, 0)),
            pl.BlockSpec((d, d), lambda i: (0, 0)),
            pl.BlockSpec((d, d), lambda i: (0, 0)),
            pl.BlockSpec((1, d), lambda i: (0, 0)),
        ],
        out_specs=[
            pl.BlockSpec((bn, d), lambda i: (i, 0)),
            pl.BlockSpec((bn, d), lambda i: (i, 0)),
        ],
        out_shape=[
            jax.ShapeDtypeStruct((n, d), jnp.uint32),
            jax.ShapeDtypeStruct((n, d), f32),
        ],
    )(user_feat, hi, w1a_t, w1b_t, b1)

    nchunks = e // (_NW * _C)
    mesh = plsc.VectorSubcoreMesh(core_axis_name="c", subcore_axis_name="s",
                                  num_cores=_NC, num_subcores=_NS)
    g1, g2 = pl.kernel(
        functools.partial(_edge_gather_body, nchunks),
        out_type=(
            jax.ShapeDtypeStruct((e, d), jnp.uint32),
            jax.ShapeDtypeStruct((e, d), f32),
        ),
        mesh=mesh,
        scratch_types=[
            pltpu.VMEM((nchunks, _C), jnp.int32),
            pltpu.VMEM((nchunks, _C), jnp.int32),
            pltpu.VMEM((2, _C, d), jnp.uint32),
            pltpu.VMEM((2, _C, d), f32),
            pltpu.SemaphoreType.DMA((2,)),
            pltpu.SemaphoreType.DMA((2,)),
            pltpu.SemaphoreType.DMA((2,)),
            pltpu.SemaphoreType.DMA((2,)),
        ],
    )(ttr_tab, b_tab, trust.reshape(_NW, nchunks, _C),
      trustee.reshape(_NW, nchunks, _C))

    be = 2560
    grid_e = e // be
    msg, ex = pl.pallas_call(
        _mlp_body,
        grid=(grid_e,),
        in_specs=[
            pl.BlockSpec((be, d), lambda i: (i, 0)),
            pl.BlockSpec((be, d), lambda i: (i, 0)),
            pl.BlockSpec((d, d), lambda i: (0, 0)),
            pl.BlockSpec((1, d), lambda i: (0, 0)),
            pl.BlockSpec((1, d), lambda i: (0, 0)),
            pl.BlockSpec((1, 1), lambda i: (0, 0)),
        ],
        out_specs=[
            pl.BlockSpec((be, d), lambda i: (i, 0)),
            pl.BlockSpec((be, 1), lambda i: (i, 0)),
        ],
        out_shape=[
            jax.ShapeDtypeStruct((e, d), f32),
            jax.ShapeDtypeStruct((e, 1), f32),
        ],
    )(g1, g2, w2t, b2, a3, b3)
    ex_flat = ex.reshape(e)

    hs_parts, ss_parts = pl.kernel(
        functools.partial(_scatter_body, nchunks, n),
        out_type=(
            jax.ShapeDtypeStruct((_NC, n, d), f32),
            jax.ShapeDtypeStruct((_NC * n,), f32),
        ),
        mesh=mesh,
        scratch_types=[
            pltpu.VMEM_SHARED((n, d), f32),
            pltpu.VMEM_SHARED((n,), f32),
            pltpu.VMEM((nchunks, _C), jnp.int32),
            pltpu.VMEM((2, _C, d), f32),
            pltpu.VMEM((nchunks * _C,), f32),
            pltpu.VMEM((1000,), f32),
            pltpu.SemaphoreType.DMA((2,)),
        ],
    )(msg, trustee.reshape(_NW, nchunks, _C), ex_flat)
    ss_parts = ss_parts.reshape(_NC, n)

    out = pl.pallas_call(
        _finish_body,
        grid=(grid_n,),
        in_specs=[
            pl.BlockSpec((bn, d), lambda i: (i, 0)),
            pl.BlockSpec((bn, d), lambda i: (i, 0)),
            pl.BlockSpec((bn, 1), lambda i: (i, 0)),
            pl.BlockSpec((bn, 1), lambda i: (i, 0)),
            pl.BlockSpec((d, d), lambda i: (0, 0)),
            pl.BlockSpec((1, d), lambda i: (0, 0)),
        ],
        out_specs=pl.BlockSpec((bn, d), lambda i: (i, 0)),
        out_shape=jax.ShapeDtypeStruct((n, d), f32),
    )(hs_parts[0], hs_parts[1], ss_parts[0][:, None], ss_parts[1][:, None],
      wwt, wb)
    return out
